# baseline (speedup 1.0000x reference)
import functools

import jax
import jax.numpy as jnp
from jax import lax
from jax.experimental import pallas as pl
from jax.experimental.pallas import tpu as pltpu
from jax.experimental.pallas import tpu_sc as plsc

VOCAB = 1000000
EMBED = 32
BATCH = 16384
N_FIELDS = 26
B = BATCH * N_FIELDS

NC = 2
NS = 16
NW = NC * NS

B_PER_W = B // NW
BATCH_PER_W = BATCH // NW
N_GROUPS = BATCH_PER_W // 16


def _make_gather():
    mesh = plsc.VectorSubcoreMesh(core_axis_name="c", subcore_axis_name="s")

    @functools.partial(
        pl.kernel,
        mesh=mesh,
        out_type=jax.ShapeDtypeStruct((N_FIELDS, 4, 128, 1024), jnp.float32),
        scratch_types=[
            pltpu.VMEM((B_PER_W,), jnp.int32),
            pltpu.VMEM((N_FIELDS, BATCH_PER_W), jnp.int32),
            pltpu.VMEM((2, BATCH_PER_W, EMBED), jnp.float32),
            pltpu.VMEM((2, 4, 4, 1024), jnp.float32),
            pltpu.SemaphoreType.DMA,
            pltpu.SemaphoreType.DMA,
            pltpu.SemaphoreType.DMA,
            pltpu.SemaphoreType.DMA,
        ],
        compiler_params=pltpu.CompilerParams(
            use_tc_tiling_on_sc=False, needs_layout_passes=False
        ),
    )
    def gather_kernel(idx_hbm, w_hbm, out_hbm, idx_all, fidx, rows_v, tbuf,
                      g0, g1, o0, o1):
        wid = lax.axis_index("s") * NC + lax.axis_index("c")
        base = wid * B_PER_W
        bb0 = wid * 4

        pltpu.sync_copy(idx_hbm.at[pl.ds(base, B_PER_W)], idx_all)

        iota16 = lax.iota(jnp.int32, 16)
        iota26 = iota16 * N_FIELDS

        @pl.loop(0, N_FIELDS)
        def _(f):
            for k in range(N_GROUPS):
                picks = plsc.load_gather(idx_all, [iota26 + (f + 26 * 16 * k)])
                fidx[f, pl.ds(16 * k, 16)] = picks

        def start_gather(f, buf, sem):
            return pltpu.async_copy(
                w_hbm.at[fidx.at[f]], rows_v.at[buf], sem
            )

        def wait_gather(f, buf, sem):
            pltpu.make_async_copy(
                w_hbm.at[fidx.at[f]], rows_v.at[buf], sem
            ).wait()

        def transpose(buf):
            rows = rows_v.at[buf]
            for k in range(N_GROUPS):
                row_ids = iota16 + (16 * k)
                for c in range(EMBED):
                    v = plsc.load_gather(
                        rows, [row_ids, jnp.full((16,), c, jnp.int32)]
                    )
                    tbuf[buf, c // 8, k // 8,
                         pl.ds((c % 8) * 128 + (k % 8) * 16, 16)] = v

        def start_out(f, buf, sem):
            return pltpu.async_copy(
                tbuf.at[buf],
                out_hbm.at[f, :, pl.ds(bb0, 4), :],
                sem,
            )

        def wait_out(f, buf, sem):
            pltpu.make_async_copy(
                tbuf.at[buf],
                out_hbm.at[f, :, pl.ds(bb0, 4), :],
                sem,
            ).wait()

        start_gather(0, 0, g0)

        @pl.loop(0, N_FIELDS // 2)
        def _(g):
            f0 = 2 * g
            f1 = f0 + 1
            start_gather(f1, 1, g1)
            wait_gather(f0, 0, g0)

            @pl.when(g > 0)
            def _():
                wait_out(f0 - 2, 0, o0)

            transpose(0)
            start_out(f0, 0, o0)

            @pl.when(g < N_FIELDS // 2 - 1)
            def _():
                start_gather(f0 + 2, 0, g0)

            wait_gather(f1, 1, g1)

            @pl.when(g > 0)
            def _():
                wait_out(f1 - 2, 1, o1)

            transpose(1)
            start_out(f1, 1, o1)

        wait_out(N_FIELDS - 2, 0, o0)
        wait_out(N_FIELDS - 1, 1, o1)

    return gather_kernel


_gather = _make_gather()


@jax.jit
def kernel(input, weight):
    idx = input.reshape(B)
    out_t = _gather(idx, weight)
    return (
        out_t.reshape(N_FIELDS, 4, 128, 8, 128)
        .transpose(2, 4, 0, 1, 3)
        .reshape(BATCH, N_FIELDS, EMBED)
    )


# speedup vs baseline: 1.1624x; 1.1624x over previous
import functools

import jax
import jax.numpy as jnp
from jax import lax
from jax.experimental import pallas as pl
from jax.experimental.pallas import tpu as pltpu
from jax.experimental.pallas import tpu_sc as plsc

VOCAB = 1000000
EMBED = 32
BATCH = 16384
N_FIELDS = 26
B = BATCH * N_FIELDS

NC = 2
NS = 16
NW = NC * NS

B_PER_W = B // NW
BATCH_PER_W = BATCH // NW
N_GROUPS = BATCH_PER_W // 16


def _make_gather():
    mesh = plsc.VectorSubcoreMesh(core_axis_name="c", subcore_axis_name="s")

    @functools.partial(
        pl.kernel,
        mesh=mesh,
        out_type=jax.ShapeDtypeStruct((N_FIELDS, 4, 128, 1024), jnp.float32),
        scratch_types=[
            pltpu.VMEM((B_PER_W,), jnp.int32),
            pltpu.VMEM((N_FIELDS, BATCH_PER_W), jnp.int32),
            pltpu.VMEM((2, BATCH_PER_W, EMBED), jnp.float32),
            pltpu.VMEM((2, 4, 4, 1024), jnp.float32),
            pltpu.SemaphoreType.DMA,
            pltpu.SemaphoreType.DMA,
            pltpu.SemaphoreType.DMA,
            pltpu.SemaphoreType.DMA,
        ],
        compiler_params=pltpu.CompilerParams(
            use_tc_tiling_on_sc=False, needs_layout_passes=False
        ),
    )
    def gather_kernel(idx_hbm, w_hbm, out_hbm, idx_all, fidx, rows_v, tbuf,
                      g0, g1, o0, o1):
        wid = lax.axis_index("s") * NC + lax.axis_index("c")
        base = wid * B_PER_W
        bb0 = wid * 4

        pltpu.sync_copy(idx_hbm.at[pl.ds(base, B_PER_W)], idx_all)

        iota16 = lax.iota(jnp.int32, 16)
        iota26 = iota16 * N_FIELDS

        @pl.loop(0, N_FIELDS)
        def _(f):
            for k in range(N_GROUPS):
                picks = plsc.load_gather(idx_all, [iota26 + (f + 26 * 16 * k)])
                fidx[f, pl.ds(16 * k, 16)] = picks

        def start_gather(f, buf, sem):
            return pltpu.async_copy(
                w_hbm.at[fidx.at[f]], rows_v.at[buf], sem
            )

        def wait_gather(f, buf, sem):
            pltpu.make_async_copy(
                w_hbm.at[fidx.at[f]], rows_v.at[buf], sem
            ).wait()

        c8_lo = lax.shift_right_logical(iota16, 3)
        c8_hi = c8_lo + 2
        pat_in = (iota16 & 7) * 128

        def transpose(buf):
            tb = tbuf.at[buf]

            @pl.loop(0, N_GROUPS)
            def _(k):
                for jq in range(16):
                    j = 16 * k + jq
                    jbb = lax.shift_right_logical(j, 7)
                    jl = j & 127
                    idx_jbb = jnp.broadcast_to(jbb, (16,))
                    idx_in = pat_in + jl
                    v_lo = rows_v[buf, j, pl.ds(0, 16)]
                    v_hi = rows_v[buf, j, pl.ds(16, 16)]
                    plsc.store_scatter(tb, [c8_lo, idx_jbb, idx_in], v_lo)
                    plsc.store_scatter(tb, [c8_hi, idx_jbb, idx_in], v_hi)

        def start_out(f, buf, sem):
            return pltpu.async_copy(
                tbuf.at[buf],
                out_hbm.at[f, :, pl.ds(bb0, 4), :],
                sem,
            )

        def wait_out(f, buf, sem):
            pltpu.make_async_copy(
                tbuf.at[buf],
                out_hbm.at[f, :, pl.ds(bb0, 4), :],
                sem,
            ).wait()

        start_gather(0, 0, g0)

        @pl.loop(0, N_FIELDS // 2)
        def _(g):
            f0 = 2 * g
            f1 = f0 + 1
            start_gather(f1, 1, g1)
            wait_gather(f0, 0, g0)

            @pl.when(g > 0)
            def _():
                wait_out(f0 - 2, 0, o0)

            transpose(0)
            start_out(f0, 0, o0)

            @pl.when(g < N_FIELDS // 2 - 1)
            def _():
                start_gather(f0 + 2, 0, g0)

            wait_gather(f1, 1, g1)

            @pl.when(g > 0)
            def _():
                wait_out(f1 - 2, 1, o1)

            transpose(1)
            start_out(f1, 1, o1)

        wait_out(N_FIELDS - 2, 0, o0)
        wait_out(N_FIELDS - 1, 1, o1)

    return gather_kernel


_gather = _make_gather()


@jax.jit
def kernel(input, weight):
    idx = input.reshape(B)
    out_t = _gather(idx, weight)
    return (
        out_t.reshape(N_FIELDS, 4, 128, 8, 128)
        .transpose(2, 4, 0, 1, 3)
        .reshape(BATCH, N_FIELDS, EMBED)
    )


# speedup vs baseline: 1.2325x; 1.0604x over previous
import functools

import jax
import jax.numpy as jnp
from jax import lax
from jax.experimental import pallas as pl
from jax.experimental.pallas import tpu as pltpu
from jax.experimental.pallas import tpu_sc as plsc

VOCAB = 1000000
EMBED = 32
BATCH = 16384
N_FIELDS = 26
B = BATCH * N_FIELDS

NC = 2
NS = 16
NW = NC * NS

B_PER_W = B // NW
BATCH_PER_W = BATCH // NW
N_GROUPS = BATCH_PER_W // 16


def _make_gather():
    mesh = plsc.VectorSubcoreMesh(core_axis_name="c", subcore_axis_name="s")

    @functools.partial(
        pl.kernel,
        mesh=mesh,
        out_type=jax.ShapeDtypeStruct((N_FIELDS, 4, 128, 1024), jnp.float32),
        scratch_types=[
            pltpu.VMEM((B_PER_W,), jnp.int32),
            pltpu.VMEM((N_FIELDS, BATCH_PER_W), jnp.int32),
            pltpu.VMEM((2, BATCH_PER_W, EMBED), jnp.float32),
            pltpu.VMEM((2, 4, 4, 1024), jnp.float32),
            pltpu.SemaphoreType.DMA,
            pltpu.SemaphoreType.DMA,
            pltpu.SemaphoreType.DMA,
            pltpu.SemaphoreType.DMA,
        ],
        compiler_params=pltpu.CompilerParams(
            use_tc_tiling_on_sc=False, needs_layout_passes=False
        ),
    )
    def gather_kernel(idx_hbm, w_hbm, out_hbm, idx_all, fidx, rows_v, tbuf,
                      g0, g1, o0, o1):
        wid = lax.axis_index("s") * NC + lax.axis_index("c")
        base = wid * B_PER_W
        bb0 = wid * 4

        pltpu.sync_copy(idx_hbm.at[pl.ds(base, B_PER_W)], idx_all)

        iota16 = lax.iota(jnp.int32, 16)
        iota26 = iota16 * N_FIELDS

        @pl.loop(0, N_FIELDS)
        def _(f):
            for k in range(N_GROUPS):
                picks = plsc.load_gather(idx_all, [iota26 + (f + 26 * 16 * k)])
                fidx[f, pl.ds(16 * k, 16)] = picks

        def start_gather(f, buf, sem):
            return pltpu.async_copy(
                w_hbm.at[fidx.at[f]], rows_v.at[buf], sem
            )

        def wait_gather(f, buf, sem):
            pltpu.make_async_copy(
                w_hbm.at[fidx.at[f]], rows_v.at[buf], sem
            ).wait()

        c8_lo = lax.shift_right_logical(iota16, 3)
        c8_hi = c8_lo + 2
        pat_in = (iota16 & 7) * 128

        def transpose(buf):
            tb = tbuf.at[buf]

            @plsc.parallel_loop(0, N_GROUPS, unroll=2)
            def _(k):
                jbb = lax.shift_right_logical(k, 3)
                jl0 = (k & 7) * 16
                idx_jbb = jnp.broadcast_to(jbb, (16,))
                base_in = pat_in + jl0
                for jq in range(16):
                    j = 16 * k + jq
                    idx_in = base_in + jq
                    v_lo = rows_v[buf, j, pl.ds(0, 16)]
                    v_hi = rows_v[buf, j, pl.ds(16, 16)]
                    plsc.store_scatter(tb, [c8_lo, idx_jbb, idx_in], v_lo)
                    plsc.store_scatter(tb, [c8_hi, idx_jbb, idx_in], v_hi)

        def start_out(f, buf, sem):
            return pltpu.async_copy(
                tbuf.at[buf],
                out_hbm.at[f, :, pl.ds(bb0, 4), :],
                sem,
            )

        def wait_out(f, buf, sem):
            pltpu.make_async_copy(
                tbuf.at[buf],
                out_hbm.at[f, :, pl.ds(bb0, 4), :],
                sem,
            ).wait()

        start_gather(0, 0, g0)

        @pl.loop(0, N_FIELDS // 2)
        def _(g):
            f0 = 2 * g
            f1 = f0 + 1
            start_gather(f1, 1, g1)
            wait_gather(f0, 0, g0)

            @pl.when(g > 0)
            def _():
                wait_out(f0 - 2, 0, o0)

            transpose(0)
            start_out(f0, 0, o0)

            @pl.when(g < N_FIELDS // 2 - 1)
            def _():
                start_gather(f0 + 2, 0, g0)

            wait_gather(f1, 1, g1)

            @pl.when(g > 0)
            def _():
                wait_out(f1 - 2, 1, o1)

            transpose(1)
            start_out(f1, 1, o1)

        wait_out(N_FIELDS - 2, 0, o0)
        wait_out(N_FIELDS - 1, 1, o1)

    return gather_kernel


_gather = _make_gather()


@jax.jit
def kernel(input, weight):
    idx = input.reshape(B)
    out_t = _gather(idx, weight)
    return (
        out_t.reshape(N_FIELDS, 4, 128, 8, 128)
        .transpose(2, 4, 0, 1, 3)
        .reshape(BATCH, N_FIELDS, EMBED)
    )
